# baseline (speedup 1.0000x reference)
import functools

import jax
import jax.numpy as jnp
from jax import lax
from jax.experimental import pallas as pl
from jax.experimental.pallas import tpu as pltpu
from jax.experimental.pallas import tpu_sc as plsc

N = 10000
E = 320000
D = 128

NC = 2
NS = 16
NW = NC * NS
EDGES_PER_TILE = E // NW
CHUNK = 64
NCHUNK = EDGES_PER_TILE // CHUNK
TAIL = EDGES_PER_TILE - NCHUNK * CHUNK
ACC_ROWS = N
NBUF = 5
ROWS_MAIN = 624
ROWS_TAIL = N - NS * ROWS_MAIN

_mesh = plsc.VectorSubcoreMesh(core_axis_name="c", subcore_axis_name="s")


def _seg_body(x_hbm, src_hbm, dst_hbm, zeros_hbm, out_hbm, acc, *bufs):
    srcs = bufs[0:2 * NBUF:2]
    dsts = bufs[1:2 * NBUF:2]
    rows = bufs[2 * NBUF:3 * NBUF]
    gsem = bufs[3 * NBUF:4 * NBUF]
    isem = bufs[4 * NBUF:5 * NBUF]
    src_t, dst_t = bufs[5 * NBUF:5 * NBUF + 2]
    c = lax.axis_index("c")
    s = lax.axis_index("s")
    wid = s * NC + c
    base = wid * EDGES_PER_TILE
    row0 = s * ROWS_MAIN

    pltpu.sync_copy(zeros_hbm.at[pl.ds(0, ROWS_MAIN)], acc.at[pl.ds(row0, ROWS_MAIN)])

    @pl.when(s == 0)
    def _():
        pltpu.sync_copy(zeros_hbm.at[pl.ds(0, ROWS_TAIL)],
                        acc.at[pl.ds(NS * ROWS_MAIN, ROWS_TAIL)])

    plsc.subcore_barrier()

    def _idx_load(i, j):
        off = base + i * CHUNK
        pltpu.sync_copy(src_hbm.at[pl.ds(off, CHUNK)], srcs[j])
        pltpu.sync_copy(dst_hbm.at[pl.ds(off, CHUNK)], dsts[j])

    def _idx_load_start(i, j):
        off = base + i * CHUNK
        pltpu.async_copy(src_hbm.at[pl.ds(off, CHUNK)], srcs[j], isem[j])
        pltpu.async_copy(dst_hbm.at[pl.ds(off, CHUNK)], dsts[j], isem[j])

    def _idx_load_wait(i, j):
        off = base + i * CHUNK
        pltpu.make_async_copy(src_hbm.at[pl.ds(off, CHUNK)], srcs[j], isem[j]).wait()
        pltpu.make_async_copy(dst_hbm.at[pl.ds(off, CHUNK)], dsts[j], isem[j]).wait()

    def _gather_start(j):
        pltpu.async_copy(x_hbm.at[srcs[j]], rows[j], gsem[j])

    def _gather_wait(j):
        pltpu.make_async_copy(x_hbm.at[srcs[j]], rows[j], gsem[j]).wait()

    def _scatter(j):
        pltpu.sync_copy(rows[j], acc.at[dsts[j]], add=True)

    for j in range(NBUF - 1):
        _idx_load(j, j)
    _idx_load_start(NBUF - 1, NBUF - 1)
    for j in range(NBUF - 1):
        _gather_start(j)

    def body(k, carry):
        cb = NBUF * k
        for j in range(NBUF):
            _gather_wait(j)
            _scatter(j)

            @pl.when(cb + j + NBUF < NCHUNK)
            def _():
                _idx_load_start(cb + j + NBUF, j)

            jn = (j + NBUF - 1) % NBUF

            @pl.when(cb + j + NBUF - 1 < NCHUNK)
            def _():
                _idx_load_wait(cb + j + NBUF - 1, jn)
                _gather_start(jn)

        return carry

    lax.fori_loop(0, NCHUNK // NBUF, body, 0)
    for t in range(NCHUNK % NBUF):
        _gather_wait(t)
        _scatter(t)

    if TAIL:
        off = base + NCHUNK * CHUNK
        pltpu.sync_copy(src_hbm.at[pl.ds(off, TAIL)], src_t)
        pltpu.sync_copy(dst_hbm.at[pl.ds(off, TAIL)], dst_t)
        pltpu.async_copy(x_hbm.at[src_t], rows[0].at[pl.ds(0, TAIL)], gsem[0]).wait()
        pltpu.sync_copy(rows[0].at[pl.ds(0, TAIL)], acc.at[dst_t], add=True)
    plsc.subcore_barrier()

    pltpu.sync_copy(acc.at[pl.ds(row0, ROWS_MAIN)],
                    out_hbm.at[c, pl.ds(row0, ROWS_MAIN)])

    @pl.when(s == 0)
    def _():
        pltpu.sync_copy(acc.at[pl.ds(NS * ROWS_MAIN, ROWS_TAIL)],
                        out_hbm.at[c, pl.ds(NS * ROWS_MAIN, ROWS_TAIL)])


_segment_sum_sc = functools.partial(
    pl.kernel,
    out_type=jax.ShapeDtypeStruct((NC, N, D), jnp.float32),
    mesh=_mesh,
    scratch_types=[
        pltpu.VMEM_SHARED((ACC_ROWS, D), jnp.float32),
    ] + [pltpu.VMEM((CHUNK,), jnp.int32) for _ in range(2 * NBUF)]
      + [pltpu.VMEM((CHUNK, D), jnp.float32) for _ in range(NBUF)]
      + [pltpu.SemaphoreType.DMA for _ in range(2 * NBUF)]
      + [pltpu.VMEM((max(TAIL, 8),), jnp.int32) for _ in range(2)],
)(_seg_body)


def _dense_tc(p, w_mu, w_rho, eps_w, b_mu, b_rho, eps_b, relu):
    blk = 1000

    def body(p_ref, wmu, wrho, ew, bmu, brho, eb, o_ref):
        w = wmu[...] + jnp.log(1.0 + jnp.exp(wrho[...])) * ew[...]
        b = bmu[...] + jnp.log(1.0 + jnp.exp(brho[...])) * eb[...]
        agg = p_ref[0] + p_ref[1]
        y = jnp.dot(agg, w, preferred_element_type=jnp.float32) + b
        if relu:
            y = jnp.maximum(y, 0.0)
        o_ref[...] = y

    return pl.pallas_call(
        body,
        grid=(N // blk,),
        in_specs=[
            pl.BlockSpec((NC, blk, D), lambda i: (0, i, 0)),
            pl.BlockSpec((D, D), lambda i: (0, 0)),
            pl.BlockSpec((D, D), lambda i: (0, 0)),
            pl.BlockSpec((D, D), lambda i: (0, 0)),
            pl.BlockSpec((1, D), lambda i: (0, 0)),
            pl.BlockSpec((1, D), lambda i: (0, 0)),
            pl.BlockSpec((1, D), lambda i: (0, 0)),
        ],
        out_specs=pl.BlockSpec((blk, D), lambda i: (i, 0)),
        out_shape=jax.ShapeDtypeStruct((N, D), jnp.float32),
    )(p, w_mu, w_rho, eps_w,
      b_mu.reshape(1, D), b_rho.reshape(1, D), eps_b.reshape(1, D))


def kernel(x, edge_index, W1_mu, W1_rho, b1_mu, b1_rho, W2_mu, W2_rho, b2_mu, b2_rho):
    src = edge_index[0]
    dst = edge_index[1]
    zeros = jnp.zeros((ROWS_MAIN, D), jnp.float32)

    k = jax.random.key(42)
    k1, k2 = jax.random.split(k)
    kW1, kb1 = jax.random.split(k1)
    kW2, kb2 = jax.random.split(k2)
    eW1 = jax.random.normal(kW1, (D, D), jnp.float32)
    eb1 = jax.random.normal(kb1, (D,), jnp.float32)
    eW2 = jax.random.normal(kW2, (D, D), jnp.float32)
    eb2 = jax.random.normal(kb2, (D,), jnp.float32)

    p1 = _segment_sum_sc(x, src, dst, zeros)
    h = _dense_tc(p1, W1_mu, W1_rho, eW1, b1_mu, b1_rho, eb1, relu=True)
    p2 = _segment_sum_sc(h, src, dst, zeros)
    out = _dense_tc(p2, W2_mu, W2_rho, eW2, b2_mu, b2_rho, eb2, relu=False)
    return out


# speedup vs baseline: 1.0118x; 1.0118x over previous
import functools

import jax
import jax.numpy as jnp
from jax import lax
from jax.experimental import pallas as pl
from jax.experimental.pallas import tpu as pltpu
from jax.experimental.pallas import tpu_sc as plsc

N = 10000
E = 320000
D = 128

NC = 2
NS = 16
NW = NC * NS
EDGES_PER_TILE = E // NW
CHUNK = 80
NCHUNK = EDGES_PER_TILE // CHUNK
TAIL = EDGES_PER_TILE - NCHUNK * CHUNK
ACC_ROWS = N
NBUF = 4
ROWS_MAIN = 624
ROWS_TAIL = N - NS * ROWS_MAIN

_mesh = plsc.VectorSubcoreMesh(core_axis_name="c", subcore_axis_name="s")


def _seg_body(x_hbm, src_hbm, dst_hbm, zeros_hbm, out_hbm,
              acc,
              src0, dst0, src1, dst1, src2, dst2, src3, dst3,
              r0, r1, r2, r3,
              g0, g1, g2, g3, i0, i1, i2_, i3):
    srcs = (src0, src1, src2, src3)
    dsts = (dst0, dst1, dst2, dst3)
    rows = (r0, r1, r2, r3)
    gsem = (g0, g1, g2, g3)
    isem = (i0, i1, i2_, i3)
    c = lax.axis_index("c")
    s = lax.axis_index("s")
    wid = s * NC + c
    base = wid * EDGES_PER_TILE
    row0 = s * ROWS_MAIN

    pltpu.sync_copy(zeros_hbm.at[pl.ds(0, ROWS_MAIN)], acc.at[pl.ds(row0, ROWS_MAIN)])

    @pl.when(s == 0)
    def _():
        pltpu.sync_copy(zeros_hbm.at[pl.ds(0, ROWS_TAIL)],
                        acc.at[pl.ds(NS * ROWS_MAIN, ROWS_TAIL)])

    plsc.subcore_barrier()

    def _idx_load(i, j):
        off = base + i * CHUNK
        pltpu.sync_copy(src_hbm.at[pl.ds(off, CHUNK)], srcs[j])
        pltpu.sync_copy(dst_hbm.at[pl.ds(off, CHUNK)], dsts[j])

    def _idx_load_start(i, j):
        off = base + i * CHUNK
        pltpu.async_copy(src_hbm.at[pl.ds(off, CHUNK)], srcs[j], isem[j])
        pltpu.async_copy(dst_hbm.at[pl.ds(off, CHUNK)], dsts[j], isem[j])

    def _idx_load_wait(i, j):
        off = base + i * CHUNK
        pltpu.make_async_copy(src_hbm.at[pl.ds(off, CHUNK)], srcs[j], isem[j]).wait()
        pltpu.make_async_copy(dst_hbm.at[pl.ds(off, CHUNK)], dsts[j], isem[j]).wait()

    def _gather_start(j):
        pltpu.async_copy(x_hbm.at[srcs[j]], rows[j], gsem[j])

    def _gather_wait(j):
        pltpu.make_async_copy(x_hbm.at[srcs[j]], rows[j], gsem[j]).wait()

    def _scatter(j):
        pltpu.sync_copy(rows[j], acc.at[dsts[j]], add=True)

    assert NCHUNK % 4 == 1
    _idx_load(0, 0)
    _idx_load(1, 1)
    _idx_load(2, 2)
    _idx_load_start(3, 3)
    _gather_start(0)
    _gather_start(1)
    _gather_start(2)

    def body(k, carry):
        cb = 4 * k
        for j in range(4):
            _gather_wait(j)
            _scatter(j)

            @pl.when(cb + j + 4 < NCHUNK)
            def _():
                _idx_load_start(cb + j + 4, j)

            jn = (j + 3) % 4

            @pl.when(cb + j + 3 < NCHUNK)
            def _():
                _idx_load_wait(cb + j + 3, jn)
                _gather_start(jn)

        return carry

    lax.fori_loop(0, NCHUNK // 4, body, 0)
    _gather_wait(0)
    _scatter(0)
    plsc.subcore_barrier()

    pltpu.sync_copy(acc.at[pl.ds(row0, ROWS_MAIN)],
                    out_hbm.at[c, pl.ds(row0, ROWS_MAIN)])

    @pl.when(s == 0)
    def _():
        pltpu.sync_copy(acc.at[pl.ds(NS * ROWS_MAIN, ROWS_TAIL)],
                        out_hbm.at[c, pl.ds(NS * ROWS_MAIN, ROWS_TAIL)])


_segment_sum_sc = functools.partial(
    pl.kernel,
    out_type=jax.ShapeDtypeStruct((NC, N, D), jnp.float32),
    mesh=_mesh,
    scratch_types=[
        pltpu.VMEM_SHARED((ACC_ROWS, D), jnp.float32),
    ] + [pltpu.VMEM((CHUNK,), jnp.int32) for _ in range(2 * NBUF)]
      + [pltpu.VMEM((CHUNK, D), jnp.float32) for _ in range(NBUF)]
      + [pltpu.SemaphoreType.DMA for _ in range(2 * NBUF)],
)(_seg_body)


def _dense_tc(p, w_mu, w_rho, eps_w, b_mu, b_rho, eps_b, relu):
    blk = 1000

    def body(p_ref, wmu, wrho, ew, bmu, brho, eb, o_ref):
        w = wmu[...] + jnp.log(1.0 + jnp.exp(wrho[...])) * ew[...]
        b = bmu[...] + jnp.log(1.0 + jnp.exp(brho[...])) * eb[...]
        agg = p_ref[0] + p_ref[1]
        y = jnp.dot(agg, w, preferred_element_type=jnp.float32) + b
        if relu:
            y = jnp.maximum(y, 0.0)
        o_ref[...] = y

    return pl.pallas_call(
        body,
        grid=(N // blk,),
        in_specs=[
            pl.BlockSpec((NC, blk, D), lambda i: (0, i, 0)),
            pl.BlockSpec((D, D), lambda i: (0, 0)),
            pl.BlockSpec((D, D), lambda i: (0, 0)),
            pl.BlockSpec((D, D), lambda i: (0, 0)),
            pl.BlockSpec((1, D), lambda i: (0, 0)),
            pl.BlockSpec((1, D), lambda i: (0, 0)),
            pl.BlockSpec((1, D), lambda i: (0, 0)),
        ],
        out_specs=pl.BlockSpec((blk, D), lambda i: (i, 0)),
        out_shape=jax.ShapeDtypeStruct((N, D), jnp.float32),
    )(p, w_mu, w_rho, eps_w,
      b_mu.reshape(1, D), b_rho.reshape(1, D), eps_b.reshape(1, D))


def kernel(x, edge_index, W1_mu, W1_rho, b1_mu, b1_rho, W2_mu, W2_rho, b2_mu, b2_rho):
    src = edge_index[0]
    dst = edge_index[1]
    zeros = jnp.zeros((ROWS_MAIN, D), jnp.float32)

    k = jax.random.key(42)
    k1, k2 = jax.random.split(k)
    kW1, kb1 = jax.random.split(k1)
    kW2, kb2 = jax.random.split(k2)
    eW1 = jax.random.normal(kW1, (D, D), jnp.float32)
    eb1 = jax.random.normal(kb1, (D,), jnp.float32)
    eW2 = jax.random.normal(kW2, (D, D), jnp.float32)
    eb2 = jax.random.normal(kb2, (D,), jnp.float32)

    p1 = _segment_sum_sc(x, src, dst, zeros)
    h = _dense_tc(p1, W1_mu, W1_rho, eW1, b1_mu, b1_rho, eb1, relu=True)
    p2 = _segment_sum_sc(h, src, dst, zeros)
    out = _dense_tc(p2, W2_mu, W2_rho, eW2, b2_mu, b2_rho, eb2, relu=False)
    return out


# speedup vs baseline: 1.0358x; 1.0237x over previous
import functools

import jax
import jax.numpy as jnp
from jax import lax
from jax.experimental import pallas as pl
from jax.experimental.pallas import tpu as pltpu
from jax.experimental.pallas import tpu_sc as plsc

N = 10000
E = 320000
D = 128

NC = 2
NS = 16
NW = NC * NS
EDGES_PER_TILE = E // NW
CHUNK = 80
NCHUNK = EDGES_PER_TILE // CHUNK
TAIL = EDGES_PER_TILE - NCHUNK * CHUNK
ACC_ROWS = N
NBUF = 4
ROWS_MAIN = 624
ROWS_TAIL = N - NS * ROWS_MAIN

_mesh = plsc.VectorSubcoreMesh(core_axis_name="c", subcore_axis_name="s")


def _seg_body(x_hbm, src_hbm, dst_hbm, zeros_hbm, out_hbm,
              acc,
              src0, dst0, src1, dst1, src2, dst2, src3, dst3,
              r0, r1, r2, r3,
              g0, g1, g2, g3, i0, i1, i2_, i3):
    srcs = (src0, src1, src2, src3)
    dsts = (dst0, dst1, dst2, dst3)
    rows = (r0, r1, r2, r3)
    gsem = (g0, g1, g2, g3)
    isem = (i0, i1, i2_, i3)
    c = lax.axis_index("c")
    s = lax.axis_index("s")
    wid = s * NC + c
    base = wid * EDGES_PER_TILE
    row0 = s * ROWS_MAIN

    pltpu.sync_copy(zeros_hbm.at[pl.ds(0, ROWS_MAIN)], acc.at[pl.ds(row0, ROWS_MAIN)])

    @pl.when(s == 0)
    def _():
        pltpu.sync_copy(zeros_hbm.at[pl.ds(0, ROWS_TAIL)],
                        acc.at[pl.ds(NS * ROWS_MAIN, ROWS_TAIL)])

    plsc.subcore_barrier()

    def _idx_load(i, j):
        off = base + i * CHUNK
        pltpu.sync_copy(src_hbm.at[pl.ds(off, CHUNK)], srcs[j])
        pltpu.sync_copy(dst_hbm.at[pl.ds(off, CHUNK)], dsts[j])

    def _idx_load_start(i, j):
        off = base + i * CHUNK
        pltpu.async_copy(src_hbm.at[pl.ds(off, CHUNK)], srcs[j], isem[j])
        pltpu.async_copy(dst_hbm.at[pl.ds(off, CHUNK)], dsts[j], isem[j])

    def _idx_load_wait(i, j):
        off = base + i * CHUNK
        pltpu.make_async_copy(src_hbm.at[pl.ds(off, CHUNK)], srcs[j], isem[j]).wait()
        pltpu.make_async_copy(dst_hbm.at[pl.ds(off, CHUNK)], dsts[j], isem[j]).wait()

    def _gather_start(j):
        pltpu.async_copy(x_hbm.at[srcs[j]], rows[j], gsem[j])

    def _gather_wait(j):
        pltpu.make_async_copy(x_hbm.at[srcs[j]], rows[j], gsem[j]).wait()

    def _scatter(j):
        pltpu.sync_copy(rows[j], acc.at[dsts[j]], add=True)

    assert NCHUNK % 4 == 1
    _idx_load(0, 0)
    _idx_load(1, 1)
    _idx_load(2, 2)
    _idx_load_start(3, 3)
    _gather_start(0)
    _gather_start(1)
    _gather_start(2)

    def body(k, carry):
        cb = 4 * k
        for j in range(4):
            _gather_wait(j)
            _scatter(j)

            @pl.when(cb + j + 4 < NCHUNK)
            def _():
                _idx_load_start(cb + j + 4, j)

            jn = (j + 3) % 4

            @pl.when(cb + j + 3 < NCHUNK)
            def _():
                _idx_load_wait(cb + j + 3, jn)
                _gather_start(jn)

        return carry

    lax.fori_loop(0, NCHUNK // 4, body, 0)
    _gather_wait(0)
    _scatter(0)
    plsc.subcore_barrier()

    pltpu.sync_copy(acc.at[pl.ds(row0, ROWS_MAIN)],
                    out_hbm.at[c, pl.ds(row0, ROWS_MAIN)])

    @pl.when(s == 0)
    def _():
        pltpu.sync_copy(acc.at[pl.ds(NS * ROWS_MAIN, ROWS_TAIL)],
                        out_hbm.at[c, pl.ds(NS * ROWS_MAIN, ROWS_TAIL)])


_segment_sum_sc = functools.partial(
    pl.kernel,
    out_type=jax.ShapeDtypeStruct((NC, N, D), jnp.float32),
    mesh=_mesh,
    scratch_types=[
        pltpu.VMEM_SHARED((ACC_ROWS, D), jnp.float32),
    ] + [pltpu.VMEM((CHUNK,), jnp.int32) for _ in range(2 * NBUF)]
      + [pltpu.VMEM((CHUNK, D), jnp.float32) for _ in range(NBUF)]
      + [pltpu.SemaphoreType.DMA for _ in range(2 * NBUF)],
)(_seg_body)


def _dense_tc(p, w_mu, w_rho, eps_w, b_mu, b_rho, eps_b, relu):
    blk = 2000

    def body(p_ref, wmu, wrho, ew, bmu, brho, eb, o_ref):
        w = wmu[...] + jnp.log(1.0 + jnp.exp(wrho[...])) * ew[...]
        b = bmu[...] + jnp.log(1.0 + jnp.exp(brho[...])) * eb[...]
        agg = p_ref[0] + p_ref[1]
        y = jnp.dot(agg, w, preferred_element_type=jnp.float32) + b
        if relu:
            y = jnp.maximum(y, 0.0)
        o_ref[...] = y

    return pl.pallas_call(
        body,
        grid=(N // blk,),
        in_specs=[
            pl.BlockSpec((NC, blk, D), lambda i: (0, i, 0)),
            pl.BlockSpec((D, D), lambda i: (0, 0)),
            pl.BlockSpec((D, D), lambda i: (0, 0)),
            pl.BlockSpec((D, D), lambda i: (0, 0)),
            pl.BlockSpec((1, D), lambda i: (0, 0)),
            pl.BlockSpec((1, D), lambda i: (0, 0)),
            pl.BlockSpec((1, D), lambda i: (0, 0)),
        ],
        out_specs=pl.BlockSpec((blk, D), lambda i: (i, 0)),
        out_shape=jax.ShapeDtypeStruct((N, D), jnp.float32),
    )(p, w_mu, w_rho, eps_w,
      b_mu.reshape(1, D), b_rho.reshape(1, D), eps_b.reshape(1, D))


def kernel(x, edge_index, W1_mu, W1_rho, b1_mu, b1_rho, W2_mu, W2_rho, b2_mu, b2_rho):
    src = edge_index[0]
    dst = edge_index[1]
    zeros = jnp.zeros((ROWS_MAIN, D), jnp.float32)

    k = jax.random.key(42)
    k1, k2 = jax.random.split(k)
    kW1, kb1 = jax.random.split(k1)
    kW2, kb2 = jax.random.split(k2)
    eW1 = jax.random.normal(kW1, (D, D), jnp.float32)
    eb1 = jax.random.normal(kb1, (D,), jnp.float32)
    eW2 = jax.random.normal(kW2, (D, D), jnp.float32)
    eb2 = jax.random.normal(kb2, (D,), jnp.float32)

    p1 = _segment_sum_sc(x, src, dst, zeros)
    h = _dense_tc(p1, W1_mu, W1_rho, eW1, b1_mu, b1_rho, eb1, relu=True)
    p2 = _segment_sum_sc(h, src, dst, zeros)
    out = _dense_tc(p2, W2_mu, W2_rho, eW2, b2_mu, b2_rho, eb2, relu=False)
    return out
